# Initial kernel scaffold; baseline (speedup 1.0000x reference)
import jax
import jax.numpy as jnp
from jax.experimental import pallas as pl


def kernel(x, wi_w, wo_w, expert_labels):
    raise NotImplementedError("write your pallas kernel here")


# speedup vs baseline: 2.5774x; 2.5774x over previous
import functools

import jax
import jax.numpy as jnp
from jax import lax
from jax.experimental import pallas as pl
from jax.experimental.pallas import tpu as pltpu

NUM_EXPERTS = 16
TOP_K = 2
D_MODEL = 2048
D_FF = 8192
TOKENS = 4096

FB1 = 512
TB2 = 512
TBB = 1024
FBB = 1024


def _mm1_body(x_ref, wi_ref, hid_ref):
    h = jnp.dot(x_ref[...], wi_ref[...].astype(jnp.bfloat16),
                preferred_element_type=jnp.float32)
    hid_ref[...] = jnp.maximum(h, 0.0).astype(jnp.bfloat16)


def _score_body(hid_ref, lab_ref, score_ref):
    eids = lax.broadcasted_iota(jnp.int32, (1, NUM_EXPERTS), 1)
    pat_t = (lab_ref[...] == eids).astype(jnp.bfloat16)
    score_ref[...] = jnp.dot(hid_ref[...], pat_t,
                             preferred_element_type=jnp.float32)


def _top2_sel(score):
    idx = lax.broadcasted_iota(jnp.int32, score.shape, 1)
    m1 = jnp.max(score, axis=1, keepdims=True)
    e1 = jnp.min(jnp.where(score == m1, idx, NUM_EXPERTS), axis=1, keepdims=True)
    score2 = jnp.where(idx == e1, -jnp.inf, score)
    m2 = jnp.max(score2, axis=1, keepdims=True)
    e2 = jnp.min(jnp.where(score2 == m2, idx, NUM_EXPERTS), axis=1, keepdims=True)
    return ((idx == e1) | (idx == e2)).astype(jnp.bfloat16)


def _mm2_body(hid_ref, score_ref, lab_ref, wo_ref, out_ref):
    j = pl.program_id(1)
    sel = _top2_sel(score_ref[...])
    eids = lax.broadcasted_iota(jnp.int32, (NUM_EXPERTS, 1), 0)
    pat = (eids == lab_ref[...]).astype(jnp.bfloat16)
    mask = jnp.dot(sel, pat, preferred_element_type=jnp.float32)
    hm = hid_ref[...] * mask.astype(jnp.bfloat16)
    part = jnp.dot(hm, wo_ref[...], preferred_element_type=jnp.float32)

    @pl.when(j == 0)
    def _():
        out_ref[...] = part

    @pl.when(j != 0)
    def _():
        out_ref[...] += part


def kernel(x, wi_w, wo_w, expert_labels):
    xt = x.reshape(TOKENS, D_MODEL).astype(jnp.bfloat16)
    labels = expert_labels.astype(jnp.int32)

    hidden = pl.pallas_call(
        _mm1_body,
        grid=(D_FF // FB1,),
        in_specs=[
            pl.BlockSpec((TOKENS, D_MODEL), lambda j: (0, 0)),
            pl.BlockSpec((D_MODEL, FB1), lambda j: (0, j)),
        ],
        out_specs=pl.BlockSpec((TOKENS, FB1), lambda j: (0, j)),
        out_shape=jax.ShapeDtypeStruct((TOKENS, D_FF), jnp.bfloat16),
        compiler_params=pltpu.CompilerParams(
            dimension_semantics=("parallel",)),
    )(xt, wi_w)

    score = pl.pallas_call(
        _score_body,
        grid=(TOKENS // TB2,),
        in_specs=[
            pl.BlockSpec((TB2, D_FF), lambda i: (i, 0)),
            pl.BlockSpec((D_FF, 1), lambda i: (0, 0)),
        ],
        out_specs=pl.BlockSpec((TB2, NUM_EXPERTS), lambda i: (i, 0)),
        out_shape=jax.ShapeDtypeStruct((TOKENS, NUM_EXPERTS), jnp.float32),
        compiler_params=pltpu.CompilerParams(
            dimension_semantics=("parallel",)),
    )(hidden, labels.reshape(D_FF, 1))

    out = pl.pallas_call(
        _mm2_body,
        grid=(TOKENS // TBB, D_FF // FBB),
        in_specs=[
            pl.BlockSpec((TBB, FBB), lambda t, j: (t, j)),
            pl.BlockSpec((TBB, NUM_EXPERTS), lambda t, j: (t, 0)),
            pl.BlockSpec((1, FBB), lambda t, j: (0, j)),
            pl.BlockSpec((FBB, D_MODEL), lambda t, j: (j, 0)),
        ],
        out_specs=pl.BlockSpec((TBB, D_MODEL), lambda t, j: (t, 0)),
        out_shape=jax.ShapeDtypeStruct((TOKENS, D_MODEL), jnp.float32),
        compiler_params=pltpu.CompilerParams(
            dimension_semantics=("parallel", "arbitrary")),
    )(hidden, score, labels.reshape(1, D_FF), wo_w.astype(jnp.bfloat16))

    return out.reshape(x.shape)


# speedup vs baseline: 2.8543x; 1.1074x over previous
import functools

import jax
import jax.numpy as jnp
from jax import lax
from jax.experimental import pallas as pl
from jax.experimental.pallas import tpu as pltpu

NUM_EXPERTS = 16
TOP_K = 2
D_MODEL = 2048
D_FF = 8192
TOKENS = 4096

FB1 = 512
TB2 = 512
TBB = 1024
FBB = 2048


def _mm1_body(x_ref, wi_ref, hid_ref):
    h = jnp.dot(x_ref[...], wi_ref[...].astype(jnp.bfloat16),
                preferred_element_type=jnp.float32)
    hid_ref[...] = jnp.maximum(h, 0.0).astype(jnp.bfloat16)


def _score_body(hid_ref, lab_ref, score_ref):
    eids = lax.broadcasted_iota(jnp.int32, (1, NUM_EXPERTS), 1)
    pat_t = (lab_ref[...] == eids).astype(jnp.bfloat16)
    score_ref[...] = jnp.dot(hid_ref[...], pat_t,
                             preferred_element_type=jnp.float32)


def _top2_idx(score):
    idx = lax.broadcasted_iota(jnp.int32, score.shape, 1)
    m1 = jnp.max(score, axis=1, keepdims=True)
    e1 = jnp.min(jnp.where(score == m1, idx, NUM_EXPERTS), axis=1, keepdims=True)
    score2 = jnp.where(idx == e1, -jnp.inf, score)
    m2 = jnp.max(score2, axis=1, keepdims=True)
    e2 = jnp.min(jnp.where(score2 == m2, idx, NUM_EXPERTS), axis=1, keepdims=True)
    return e1, e2


def _mm2_body(hid_ref, score_ref, lab_ref, wo_ref, out_ref):
    j = pl.program_id(1)
    e1, e2 = _top2_idx(score_ref[...])
    lab = lab_ref[...]
    keep = (lab == e1) | (lab == e2)
    hm = jnp.where(keep, hid_ref[...], jnp.bfloat16(0))
    part = jnp.dot(hm, wo_ref[...], preferred_element_type=jnp.float32)

    @pl.when(j == 0)
    def _():
        out_ref[...] = part

    @pl.when(j != 0)
    def _():
        out_ref[...] += part


def kernel(x, wi_w, wo_w, expert_labels):
    xt = x.reshape(TOKENS, D_MODEL).astype(jnp.bfloat16)
    labels = expert_labels.astype(jnp.int32)

    hidden = pl.pallas_call(
        _mm1_body,
        grid=(D_FF // FB1,),
        in_specs=[
            pl.BlockSpec((TOKENS, D_MODEL), lambda j: (0, 0)),
            pl.BlockSpec((D_MODEL, FB1), lambda j: (0, j)),
        ],
        out_specs=pl.BlockSpec((TOKENS, FB1), lambda j: (0, j)),
        out_shape=jax.ShapeDtypeStruct((TOKENS, D_FF), jnp.bfloat16),
        compiler_params=pltpu.CompilerParams(
            dimension_semantics=("parallel",)),
    )(xt, wi_w)

    score = pl.pallas_call(
        _score_body,
        grid=(TOKENS // TB2,),
        in_specs=[
            pl.BlockSpec((TB2, D_FF), lambda i: (i, 0)),
            pl.BlockSpec((D_FF, 1), lambda i: (0, 0)),
        ],
        out_specs=pl.BlockSpec((TB2, NUM_EXPERTS), lambda i: (i, 0)),
        out_shape=jax.ShapeDtypeStruct((TOKENS, NUM_EXPERTS), jnp.float32),
        compiler_params=pltpu.CompilerParams(
            dimension_semantics=("parallel",)),
    )(hidden, labels.reshape(D_FF, 1))

    out = pl.pallas_call(
        _mm2_body,
        grid=(TOKENS // TBB, D_FF // FBB),
        in_specs=[
            pl.BlockSpec((TBB, FBB), lambda t, j: (t, j)),
            pl.BlockSpec((TBB, NUM_EXPERTS), lambda t, j: (t, 0)),
            pl.BlockSpec((1, FBB), lambda t, j: (0, j)),
            pl.BlockSpec((FBB, D_MODEL), lambda t, j: (j, 0)),
        ],
        out_specs=pl.BlockSpec((TBB, D_MODEL), lambda t, j: (t, 0)),
        out_shape=jax.ShapeDtypeStruct((TOKENS, D_MODEL), jnp.float32),
        compiler_params=pltpu.CompilerParams(
            dimension_semantics=("parallel", "arbitrary")),
    )(hidden, score, labels.reshape(1, D_FF), wo_w.astype(jnp.bfloat16))

    return out.reshape(x.shape)
